# baseline (speedup 1.0000x reference)
import functools

import jax
import jax.numpy as jnp
from jax import lax
from jax.experimental import pallas as pl
from jax.experimental.pallas import tpu as pltpu
from jax.experimental.pallas import tpu_sc as plsc

BATCH = 16384
HIST = 50
EMBED_DIM = 64
VOCAB = 1000000
NB = BATCH * HIST
NW = 32
CB = 128
BPW = BATCH // NW
CPW = BPW // CB
UNITS = HIST * CPW
NROWS = 3
NOUT = 2


def _make_kernel():
  mesh = plsc.VectorSubcoreMesh(core_axis_name="c", subcore_axis_name="s")

  @functools.partial(
      pl.kernel,
      out_type=jax.ShapeDtypeStruct((HIST, EMBED_DIM, BATCH), jnp.float32),
      name="embed_gather_t",
      mesh=mesh,
      scratch_types=[
          pltpu.VMEM((CB,), jnp.int32),
          pltpu.VMEM((CB,), jnp.int32),
          pltpu.VMEM((CB,), jnp.int32),
          pltpu.VMEM((CB, 128), jnp.float32),
          pltpu.VMEM((CB, 128), jnp.float32),
          pltpu.VMEM((CB, 128), jnp.float32),
          pltpu.VMEM((EMBED_DIM, CB + 1), jnp.float32),
          pltpu.VMEM((EMBED_DIM, CB + 1), jnp.float32),
          pltpu.SemaphoreType.DMA((NROWS,)),
          pltpu.SemaphoreType.DMA((NROWS,)),
          pltpu.SemaphoreType.DMA((NOUT,)),
      ],
      compiler_params=pltpu.CompilerParams(
          use_tc_tiling_on_sc=True, needs_layout_passes=False),
  )
  def gather_kernel(idx_hbm, table_hbm, out_hbm, idx0, idx1, idx2, rows0,
                    rows1, rows2, outt0, outt1, idx_sem, gat_sem, st_sem):
    idx_v = [idx0, idx1, idx2]
    rows_v = [rows0, rows1, rows2]
    outt_v = [outt0, outt1]
    wid = lax.axis_index("s") * 2 + lax.axis_index("c")
    b_base = wid * BPW

    def unit_coords(u):
      h = u // CPW
      b0 = b_base + (u % CPW) * CB
      return h, b0

    def issue_idx(u, s):
      h, b0 = unit_coords(u)
      pltpu.async_copy(
          idx_hbm.at[pl.ds(h * BATCH + b0, CB)], idx_v[s], idx_sem.at[s])

    def wait_idx(u, s):
      h, b0 = unit_coords(u)
      pltpu.make_async_copy(
          idx_hbm.at[pl.ds(h * BATCH + b0, CB)], idx_v[s],
          idx_sem.at[s]).wait()

    def issue_gather(s):
      pltpu.async_copy(table_hbm.at[idx_v[s]], rows_v[s], gat_sem.at[s])

    def wait_gather(s):
      pltpu.make_async_copy(
          table_hbm.at[idx_v[s]], rows_v[s], gat_sem.at[s]).wait()

    def issue_store(u, t):
      h, b0 = unit_coords(u)
      pltpu.async_copy(
          outt_v[t].at[:, pl.ds(0, CB)], out_hbm.at[h, :, pl.ds(b0, CB)],
          st_sem.at[t])

    def wait_store(u, t):
      h, b0 = unit_coords(u)
      pltpu.make_async_copy(
          outt_v[t].at[:, pl.ds(0, CB)], out_hbm.at[h, :, pl.ds(b0, CB)],
          st_sem.at[t]).wait()

    iota = lax.iota(jnp.int32, 16)

    def transpose_unit(s, t):
      rows = rows_v[s]
      outt = outt_v[t]

      def bloop(g, carry):
        for jb in range(8):
          b = g * 8 + jb
          idx_b = jnp.full((16,), 0, jnp.int32) + b
          xs = [rows[b, pl.ds(k * 16, 16)] for k in range(4)]
          for k in range(4):
            plsc.store_scatter(outt, [iota + (k * 16), idx_b], xs[k])
        return carry

      lax.fori_loop(0, CB // 8, bloop, 0)

    def visit(u, s, t, first, last, refill=True):
      wait_gather(s)
      if not last:
        if refill:
          issue_idx(u + NROWS, s)
        wait_idx(u + 2, (s + 2) % NROWS)
        issue_gather((s + 2) % NROWS)
      if not first:
        wait_store(u - NOUT, t)
      transpose_unit(s, t)
      issue_store(u, t)

    issue_idx(0, 0)
    issue_idx(1, 1)
    issue_idx(2, 2)
    wait_idx(0, 0)
    issue_gather(0)
    wait_idx(1, 1)
    issue_gather(1)

    visit(0, 0, 0, True, False)
    visit(1, 1, 1, True, False)

    def body(g, carry):
      for j in range(6):
        visit(2 + g * 6 + j, (2 + j) % NROWS, j % NOUT, False, False)
      return carry

    lax.fori_loop(0, (UNITS - 8) // 6, body, 0)

    for u in range(UNITS - 6, UNITS - 3):
      visit(u, u % NROWS, u % NOUT, False, False)
    visit(UNITS - 3, (UNITS - 3) % NROWS, (UNITS - 3) % NOUT, False, False,
          refill=False)
    visit(UNITS - 2, (UNITS - 2) % NROWS, (UNITS - 2) % NOUT, False, True)
    visit(UNITS - 1, (UNITS - 1) % NROWS, (UNITS - 1) % NOUT, False, True)
    wait_store(UNITS - 2, (UNITS - 2) % NOUT)
    wait_store(UNITS - 1, (UNITS - 1) % NOUT)

  return gather_kernel


_gather = _make_kernel()


@jax.jit
def kernel(inputs, table):
  tbl = jnp.pad(table, ((0, 0), (0, 128 - EMBED_DIM)))
  flat_idx = jnp.transpose(inputs).reshape(NB).astype(jnp.int32)
  out = _gather(flat_idx, tbl)
  return jnp.transpose(out, (2, 0, 1))


# speedup vs baseline: 1.3175x; 1.3175x over previous
import functools

import jax
import jax.numpy as jnp
from jax import lax
from jax.experimental import pallas as pl
from jax.experimental.pallas import tpu as pltpu
from jax.experimental.pallas import tpu_sc as plsc

BATCH = 16384
HIST = 50
EMBED_DIM = 64
VOCAB = 1000000
NB = BATCH * HIST
NW = 32
B_PER_W = NB // NW
CHUNK = 400
NBUF = 4
N_CHUNKS = B_PER_W // CHUNK
GROUPS = N_CHUNKS // NBUF


def _make_kernel():
  mesh = plsc.VectorSubcoreMesh(core_axis_name="c", subcore_axis_name="s")

  @functools.partial(
      pl.kernel,
      out_type=jax.ShapeDtypeStruct((NB, EMBED_DIM), jnp.float32),
      name="embed_gather",
      mesh=mesh,
      scratch_types=[
          pltpu.VMEM((CHUNK,), jnp.int32),
          pltpu.VMEM((CHUNK,), jnp.int32),
          pltpu.VMEM((CHUNK,), jnp.int32),
          pltpu.VMEM((CHUNK,), jnp.int32),
          pltpu.VMEM((NBUF, CHUNK, EMBED_DIM), jnp.float32),
          pltpu.SemaphoreType.DMA((NBUF,)),
          pltpu.SemaphoreType.DMA((NBUF,)),
          pltpu.SemaphoreType.DMA((NBUF,)),
      ],
      compiler_params=pltpu.CompilerParams(use_tc_tiling_on_sc=False),
  )
  def gather_kernel(idx_hbm, table_hbm, out_hbm, idx0, idx1, idx2, idx3,
                    rows_v, idx_sem, gat_sem, st_sem):
    idx_v = [idx0, idx1, idx2, idx3]
    wid = lax.axis_index("s") * 2 + lax.axis_index("c")
    w_base = wid * B_PER_W

    def issue_idx(ci, b):
      pltpu.async_copy(
          idx_hbm.at[pl.ds(w_base + ci * CHUNK, CHUNK)], idx_v[b],
          idx_sem.at[b])

    def wait_idx(ci, b):
      pltpu.make_async_copy(
          idx_hbm.at[pl.ds(w_base + ci * CHUNK, CHUNK)], idx_v[b],
          idx_sem.at[b]).wait()

    def issue_gather(b):
      pltpu.async_copy(table_hbm.at[idx_v[b]], rows_v.at[b], gat_sem.at[b])

    def wait_gather(b):
      pltpu.make_async_copy(
          table_hbm.at[idx_v[b]], rows_v.at[b], gat_sem.at[b]).wait()

    def issue_store(ci, b):
      pltpu.async_copy(
          rows_v.at[b], out_hbm.at[pl.ds(w_base + ci * CHUNK, CHUNK)],
          st_sem.at[b])

    def wait_store(ci, b):
      pltpu.make_async_copy(
          rows_v.at[b], out_hbm.at[pl.ds(w_base + ci * CHUNK, CHUNK)],
          st_sem.at[b]).wait()

    for b in range(NBUF):
      issue_idx(b, b)
    wait_idx(0, 0)
    issue_gather(0)
    wait_idx(1, 1)
    issue_gather(1)
    for b in range(NBUF):
      wait_gather(b)
      issue_store(b, b)
      issue_idx(b + NBUF, b)
      bj = (b + 2) % NBUF
      if b >= 2:
        wait_store(b - 2, bj)
      wait_idx(b + 2, bj)
      issue_gather(bj)

    def body(g, carry):
      for b in range(NBUF):
        ci = g * NBUF + b
        wait_gather(b)
        issue_store(ci, b)
        issue_idx(ci + NBUF, b)
        bj = (b + 2) % NBUF
        wait_store(ci - 2, bj)
        wait_idx(ci + 2, bj)
        issue_gather(bj)
      return carry

    lax.fori_loop(1, GROUPS - 1, body, 0)

    for b in range(NBUF):
      ci = (GROUPS - 1) * NBUF + b
      wait_gather(b)
      issue_store(ci, b)
      bj = (b + 2) % NBUF
      wait_store(ci - 2, bj)
      if ci + 2 < N_CHUNKS:
        wait_idx(ci + 2, bj)
        issue_gather(bj)
    wait_store(N_CHUNKS - 2, (N_CHUNKS - 2) % NBUF)
    wait_store(N_CHUNKS - 1, (N_CHUNKS - 1) % NBUF)

  return gather_kernel


_gather = _make_kernel()


@jax.jit
def kernel(inputs, table):
  tbl = jnp.pad(table, ((0, 0), (0, 128 - EMBED_DIM)))
  tbl_v = tbl.reshape(2 * VOCAB, EMBED_DIM)
  flat_idx = inputs.reshape(NB).astype(jnp.int32) * 2
  out = _gather(flat_idx, tbl_v)
  return out.reshape(BATCH, HIST, EMBED_DIM)
